# Initial kernel scaffold; baseline (speedup 1.0000x reference)
import jax
import jax.numpy as jnp
from jax.experimental import pallas as pl


def kernel(x, table, W, b):
    raise NotImplementedError("write your pallas kernel here")


# speedup vs baseline: 10.5412x; 10.5412x over previous
import functools

import jax
import jax.numpy as jnp
from jax import lax
from jax.experimental import pallas as pl
from jax.experimental.pallas import tpu as pltpu
from jax.experimental.pallas import tpu_sc as plsc

CDIM = 16
EDIM = 64
PACK = 8


@functools.partial(jax.jit, static_argnames=("n_rows",))
def _sc_gather(idx, table, n_rows):
    info = plsc.get_sparse_core_info()
    nw = info.num_cores * info.num_subcores
    per_w = n_rows // nw
    n_chunks = 8
    chunk = per_w // n_chunks
    mesh = plsc.VectorSubcoreMesh(core_axis_name="c", subcore_axis_name="s")

    @functools.partial(
        pl.kernel,
        mesh=mesh,
        out_type=jax.ShapeDtypeStruct((n_rows, CDIM), jnp.float32),
        scratch_types=[
            pltpu.VMEM((chunk,), jnp.int32),
            pltpu.VMEM((chunk, CDIM), jnp.float32),
            pltpu.SemaphoreType.DMA,
        ],
        compiler_params=pltpu.CompilerParams(use_tc_tiling_on_sc=False),
    )
    def gather(idx_hbm, table_hbm, out_hbm, idx_v, rows_v, sem):
        wid = lax.axis_index("s") * info.num_cores + lax.axis_index("c")
        base0 = wid * per_w
        for j in range(n_chunks):
            base = base0 + j * chunk
            pltpu.sync_copy(idx_hbm.at[pl.ds(base, chunk)], idx_v)
            pltpu.async_copy(table_hbm.at[idx_v], rows_v, sem).wait()
            pltpu.sync_copy(rows_v, out_hbm.at[pl.ds(base, chunk)])

    return gather(idx, table)


def _mm_body(emb_ref, w_ref, b_ref, out_ref):
    out_ref[...] = (
        jnp.dot(emb_ref[...], w_ref[...], preferred_element_type=jnp.float32)
        + b_ref[...]
    )


def _tc_project(emb_packed, w8, b8):
    n = emb_packed.shape[0]
    blk = 512
    return pl.pallas_call(
        _mm_body,
        grid=(n // blk,),
        in_specs=[
            pl.BlockSpec((blk, PACK * CDIM), lambda i: (i, 0)),
            pl.BlockSpec((PACK * CDIM, PACK * EDIM), lambda i: (0, 0)),
            pl.BlockSpec((1, PACK * EDIM), lambda i: (0, 0)),
        ],
        out_specs=pl.BlockSpec((blk, PACK * EDIM), lambda i: (i, 0)),
        out_shape=jax.ShapeDtypeStruct((n, PACK * EDIM), jnp.float32),
    )(emb_packed, w8, b8)


def kernel(x, table, W, b):
    batch, feat = x.shape
    n_rows = batch * feat
    idx = x.reshape(-1).astype(jnp.int32)
    emb = _sc_gather(idx, table, n_rows)
    w8 = jnp.kron(jnp.eye(PACK, dtype=jnp.float32), W.T)
    b8 = jnp.tile(b, PACK)[None, :]
    out = _tc_project(emb.reshape(n_rows // PACK, PACK * CDIM), w8, b8)
    return out.reshape(batch, feat, EDIM)


# speedup vs baseline: 12.3487x; 1.1715x over previous
import functools

import jax
import jax.numpy as jnp
from jax import lax
from jax.experimental import pallas as pl
from jax.experimental.pallas import tpu as pltpu
from jax.experimental.pallas import tpu_sc as plsc

CDIM = 16
EDIM = 64
NB = 2048
FGRP = 13


@jax.jit
def _sc_gather(xT, table):
    nf, nb = xT.shape
    info = plsc.get_sparse_core_info()
    nw = info.num_cores * info.num_subcores
    fg = 2
    bg = nw // fg
    f_per = nf // fg
    b_per = nb // bg
    mesh = plsc.VectorSubcoreMesh(core_axis_name="c", subcore_axis_name="s")

    @functools.partial(
        pl.kernel,
        mesh=mesh,
        out_type=jax.ShapeDtypeStruct((nb // 128, nf * CDIM, 128), jnp.float32),
        scratch_types=[
            pltpu.VMEM((f_per, b_per), jnp.int32),
            pltpu.VMEM((2, b_per, CDIM), jnp.float32),
            pltpu.VMEM((2, b_per // 128, CDIM, 128), jnp.float32),
            pltpu.SemaphoreType.DMA,
            pltpu.SemaphoreType.DMA,
            pltpu.SemaphoreType.DMA,
            pltpu.SemaphoreType.DMA,
        ],
        compiler_params=pltpu.CompilerParams(
            use_tc_tiling_on_sc=False, needs_layout_passes=False
        ),
    )
    def gather(xT_hbm, table_hbm, out_hbm, idx_v, rows_v, xpose_v,
               g0, g1, s0, s1):
        wid = lax.axis_index("s") * info.num_cores + lax.axis_index("c")
        f0 = (wid % fg) * f_per
        b0 = (wid // fg) * b_per
        gsem = (g0, g1)
        ssem = (s0, s1)
        pltpu.sync_copy(
            xT_hbm.at[pl.ds(f0, f_per), pl.ds(b0, b_per)], idx_v
        )
        g = [None] * f_per
        s = [None] * f_per

        def fire_gather(j):
            g[j] = pltpu.async_copy(
                table_hbm.at[idx_v.at[j]], rows_v.at[j % 2], gsem[j % 2]
            )

        def transpose_chunk(j):
            rows_ref = rows_v.at[j % 2]
            xp_ref = xpose_v.at[j % 2]

            def body(grp, carry):
                base = grp * CDIM
                ridx = lax.iota(jnp.int32, 16) + base
                q = grp // (128 // CDIM)
                lane0 = (grp % (128 // CDIM)) * CDIM
                for c in range(CDIM):
                    cidx = jnp.full((16,), c, jnp.int32)
                    xp_ref[q, c, pl.ds(lane0, CDIM)] = plsc.load_gather(
                        rows_ref, [ridx, cidx]
                    )
                return carry

            lax.fori_loop(0, b_per // CDIM, body, 0)

        def fire_scatter(j):
            s[j] = pltpu.async_copy(
                xpose_v.at[j % 2],
                out_hbm.at[
                    pl.ds(b0 // 128, b_per // 128),
                    pl.ds((f0 + j) * CDIM, CDIM),
                    :,
                ],
                ssem[j % 2],
            )

        fire_gather(0)
        for j in range(f_per):
            g[j].wait()
            if j + 1 < f_per:
                fire_gather(j + 1)
            if j >= 2:
                s[j - 2].wait()
            transpose_chunk(j)
            fire_scatter(j)
        s[f_per - 1].wait()
        s[f_per - 2].wait()

    return gather(xT, table)


def _mm_body(x_ref, w_ref, b_ref, out_ref):
    mm = lax.dot_general(
        w_ref[...], x_ref[0], (((1,), (0,)), ((), ())),
        preferred_element_type=jnp.float32,
    )
    out_ref[...] = mm.reshape(FGRP, EDIM, 128) + b_ref[...]


def _tc_project(x_cmaj, w_big, b3):
    nq, nrow, _ = x_cmaj.shape
    nf = nrow // CDIM
    nbatch = nq * 128
    return pl.pallas_call(
        _mm_body,
        grid=(nf // FGRP, nq),
        in_specs=[
            pl.BlockSpec((1, FGRP * CDIM, 128), lambda g, q: (q, g, 0)),
            pl.BlockSpec((FGRP * EDIM, FGRP * CDIM), lambda g, q: (0, 0)),
            pl.BlockSpec((1, EDIM, 1), lambda g, q: (0, 0, 0)),
        ],
        out_specs=pl.BlockSpec((FGRP, EDIM, 128), lambda g, q: (g, 0, q)),
        out_shape=jax.ShapeDtypeStruct((nf, EDIM, nbatch), jnp.float32),
    )(x_cmaj, w_big, b3)


def kernel(x, table, W, b):
    batch, feat = x.shape
    xT = x.T
    x_cmaj = _sc_gather(xT, table)
    w_big = jnp.kron(jnp.eye(FGRP, dtype=jnp.float32), W)
    outp = _tc_project(x_cmaj, w_big, b.reshape(1, EDIM, 1))
    return jnp.transpose(outp, (2, 0, 1))


# speedup vs baseline: 13.4032x; 1.0854x over previous
import functools

import jax
import jax.numpy as jnp
from jax import lax
from jax.experimental import pallas as pl
from jax.experimental.pallas import tpu as pltpu
from jax.experimental.pallas import tpu_sc as plsc

CDIM = 16
EDIM = 64
NB = 2048
FGRP = 13


@jax.jit
def _sc_gather(xT, table):
    nf, nb = xT.shape
    info = plsc.get_sparse_core_info()
    nw = info.num_cores * info.num_subcores
    fg = 2
    bg = nw // fg
    f_per = nf // fg
    b_per = nb // bg
    mesh = plsc.VectorSubcoreMesh(core_axis_name="c", subcore_axis_name="s")

    @functools.partial(
        pl.kernel,
        mesh=mesh,
        out_type=jax.ShapeDtypeStruct((nb // 128, nf * CDIM, 128), jnp.float32),
        scratch_types=[
            pltpu.VMEM((f_per, b_per), jnp.int32),
            pltpu.VMEM((2, b_per, CDIM), jnp.float32),
            pltpu.VMEM((2, b_per // 128, CDIM, 128), jnp.float32),
            pltpu.SemaphoreType.DMA,
            pltpu.SemaphoreType.DMA,
            pltpu.SemaphoreType.DMA,
            pltpu.SemaphoreType.DMA,
        ],
        compiler_params=pltpu.CompilerParams(
            use_tc_tiling_on_sc=False, needs_layout_passes=False
        ),
    )
    def gather(xT_hbm, table_hbm, out_hbm, idx_v, rows_v, xpose_v,
               g0, g1, s0, s1):
        wid = lax.axis_index("s") * info.num_cores + lax.axis_index("c")
        f0 = (wid % fg) * f_per
        b0 = (wid // fg) * b_per
        gsem = (g0, g1)
        ssem = (s0, s1)
        pltpu.sync_copy(
            xT_hbm.at[pl.ds(f0, f_per), pl.ds(b0, b_per)], idx_v
        )
        g = [None] * f_per
        s = [None] * f_per

        def fire_gather(j):
            g[j] = pltpu.async_copy(
                table_hbm.at[idx_v.at[j]], rows_v.at[j % 2], gsem[j % 2]
            )

        def transpose_chunk(j):
            rows_ref = rows_v.at[j % 2]
            xp_ref = xpose_v.at[j % 2]

            def body(grp, carry):
                base = grp * CDIM
                ridx = lax.iota(jnp.int32, 16) + base
                q = grp // (128 // CDIM)
                lane0 = (grp % (128 // CDIM)) * CDIM
                for c in range(CDIM):
                    cidx = jnp.full((16,), c, jnp.int32)
                    xp_ref[q, c, pl.ds(lane0, CDIM)] = plsc.load_gather(
                        rows_ref, [ridx, cidx]
                    )
                return carry

            lax.fori_loop(0, b_per // CDIM, body, 0)

        def fire_scatter(j):
            s[j] = pltpu.async_copy(
                xpose_v.at[j % 2],
                out_hbm.at[
                    pl.ds(b0 // 128, b_per // 128),
                    pl.ds((f0 + j) * CDIM, CDIM),
                    :,
                ],
                ssem[j % 2],
            )

        fire_gather(0)
        for j in range(f_per):
            g[j].wait()
            if j + 1 < f_per:
                fire_gather(j + 1)
            if j >= 2:
                s[j - 2].wait()
            transpose_chunk(j)
            fire_scatter(j)
        s[f_per - 1].wait()
        s[f_per - 2].wait()

    return gather(xT, table)


def _mm_body(x_ref, w_ref, b_ref, out_ref):
    nf = out_ref.shape[0]
    mm = lax.dot_general(
        w_ref[...], x_ref[0], (((1,), (0,)), ((), ())),
        preferred_element_type=jnp.float32,
    )
    out_ref[...] = mm.reshape(nf, EDIM, 128) + b_ref[...]


def _tc_project(x_cmaj, w_big, b3):
    nq, nrow, _ = x_cmaj.shape
    nf = nrow // CDIM
    nbatch = nq * 128
    return pl.pallas_call(
        _mm_body,
        grid=(nq,),
        in_specs=[
            pl.BlockSpec((1, nrow, 128), lambda q: (q, 0, 0)),
            pl.BlockSpec((nf * EDIM, nrow), lambda q: (0, 0)),
            pl.BlockSpec((1, EDIM, 1), lambda q: (0, 0, 0)),
        ],
        out_specs=pl.BlockSpec((nf, EDIM, 128), lambda q: (0, 0, q)),
        out_shape=jax.ShapeDtypeStruct((nf, EDIM, nbatch), jnp.float32),
    )(x_cmaj, w_big, b3)


def kernel(x, table, W, b):
    batch, feat = x.shape
    xT = x.T
    x_cmaj = _sc_gather(xT, table)
    w_big = jnp.kron(jnp.eye(feat, dtype=jnp.float32), W)
    outp = _tc_project(x_cmaj, w_big, b.reshape(1, EDIM, 1))
    return jnp.transpose(outp, (2, 0, 1))


# speedup vs baseline: 24.9895x; 1.8644x over previous
import functools

import jax
import jax.numpy as jnp
from jax import lax
from jax.experimental import pallas as pl
from jax.experimental.pallas import tpu as pltpu
from jax.experimental.pallas import tpu_sc as plsc

CDIM = 16
EDIM = 64
NB = 2048
FGRP = 13


@jax.jit
def _sc_linearize(tT):
    cd, nrows = tT.shape
    info = plsc.get_sparse_core_info()
    nw = info.num_cores * info.num_subcores
    ch = 512
    per_w = 31232
    nch = per_w // ch
    tail = nrows - nw * per_w
    mesh = plsc.VectorSubcoreMesh(core_axis_name="c", subcore_axis_name="s")

    @functools.partial(
        pl.kernel,
        mesh=mesh,
        out_type=jax.ShapeDtypeStruct((nrows * cd,), jnp.float32),
        scratch_types=[
            pltpu.VMEM((cd, ch), jnp.float32),
            pltpu.VMEM((cd, ch), jnp.float32),
            pltpu.VMEM((ch * cd,), jnp.float32),
            pltpu.VMEM((ch * cd,), jnp.float32),
            pltpu.SemaphoreType.DMA,
            pltpu.SemaphoreType.DMA,
            pltpu.SemaphoreType.DMA,
            pltpu.SemaphoreType.DMA,
        ],
        compiler_params=pltpu.CompilerParams(
            use_tc_tiling_on_sc=True, needs_layout_passes=False
        ),
    )
    def lin(t_hbm, out_hbm, ibuf0, ibuf1, obuf0, obuf1, g0, g1, s0, s1):
        wid = lax.axis_index("s") * info.num_cores + lax.axis_index("c")
        base = wid * per_w
        ib = (ibuf0, ibuf1)
        ob = (obuf0, obuf1)
        gsem = (g0, g1)
        ssem = (s0, s1)
        lanes = lax.iota(jnp.int32, 16)

        def in_cp(j, p):
            return pltpu.make_async_copy(
                t_hbm.at[:, pl.ds(pl.multiple_of(base + j * ch, 128), ch)],
                ib[p], gsem[p],
            )

        def out_cp(j, p):
            return pltpu.make_async_copy(
                ob[p],
                out_hbm.at[pl.ds(pl.multiple_of((base + j * ch) * cd, 8),
                                 ch * cd)],
                ssem[p],
            )

        def repack(p, ngrp):
            src, dst = ib[p], ob[p]

            def body(g, carry):
                rl0 = g * 16
                for c in range(cd):
                    v = src[c, pl.ds(rl0, 16)]
                    idx = lanes * cd + (rl0 * cd + c)
                    plsc.store_scatter(dst, [idx], v)
                return carry

            lax.fori_loop(0, ngrp, body, 0)

        in_cp(0, 0).start()
        in_cp(1, 1).start()

        def chunk_pair(t, carry):
            for p in range(2):
                j = 2 * t + p

                @pl.when(j < nch)
                def _():
                    in_cp(j, p).wait()

                    @pl.when(t >= 1)
                    def _():
                        out_cp(j - 2, p).wait()

                    repack(p, ch // 16)
                    out_cp(j, p).start()

                    @pl.when(j + 2 < nch)
                    def _():
                        in_cp(j + 2, p).start()

            return carry

        lax.fori_loop(0, (nch + 1) // 2, chunk_pair, 0)
        out_cp(nch - 2, (nch - 2) % 2).wait()
        out_cp(nch - 1, (nch - 1) % 2).wait()

        @pl.when(wid == nw - 1)
        def _():
            tb = nw * per_w
            pltpu.sync_copy(t_hbm.at[:, pl.ds(tb, ch)], ib[0])
            repack(0, ch // 16)
            pltpu.sync_copy(ob[0], out_hbm.at[pl.ds(tb * cd, ch * cd)])

    return lin(tT)


@jax.jit
def _sc_gather(xT, table):
    nf, nb = xT.shape
    info = plsc.get_sparse_core_info()
    nw = info.num_cores * info.num_subcores
    fg = 2
    bg = nw // fg
    f_per = nf // fg
    b_per = nb // bg
    mesh = plsc.VectorSubcoreMesh(core_axis_name="c", subcore_axis_name="s")

    @functools.partial(
        pl.kernel,
        mesh=mesh,
        out_type=jax.ShapeDtypeStruct((nb // 128, nf * CDIM, 128), jnp.float32),
        scratch_types=[
            pltpu.VMEM((f_per, b_per), jnp.int32),
            pltpu.VMEM((2, b_per, CDIM), jnp.float32),
            pltpu.VMEM((2, b_per // 128, CDIM, 128), jnp.float32),
            pltpu.SemaphoreType.DMA,
            pltpu.SemaphoreType.DMA,
            pltpu.SemaphoreType.DMA,
            pltpu.SemaphoreType.DMA,
        ],
        compiler_params=pltpu.CompilerParams(
            use_tc_tiling_on_sc=False, needs_layout_passes=False
        ),
    )
    def gather(xT_hbm, table_hbm, out_hbm, idx_v, rows_v, xpose_v,
               g0, g1, s0, s1):
        wid = lax.axis_index("s") * info.num_cores + lax.axis_index("c")
        f0 = (wid % fg) * f_per
        b0 = (wid // fg) * b_per
        gsem = (g0, g1)
        ssem = (s0, s1)
        pltpu.sync_copy(
            xT_hbm.at[pl.ds(f0, f_per), pl.ds(b0, b_per)], idx_v
        )
        g = [None] * f_per
        s = [None] * f_per

        def fire_gather(j):
            g[j] = pltpu.async_copy(
                table_hbm.at[idx_v.at[j]], rows_v.at[j % 2], gsem[j % 2]
            )

        def transpose_chunk(j):
            rows_ref = rows_v.at[j % 2]
            xp_ref = xpose_v.at[j % 2]

            def body(grp, carry):
                base = grp * CDIM
                ridx = lax.iota(jnp.int32, 16) + base
                q = grp // (128 // CDIM)
                lane0 = (grp % (128 // CDIM)) * CDIM
                for c in range(CDIM):
                    cidx = jnp.full((16,), c, jnp.int32)
                    xp_ref[q, c, pl.ds(lane0, CDIM)] = plsc.load_gather(
                        rows_ref, [ridx, cidx]
                    )
                return carry

            lax.fori_loop(0, b_per // CDIM, body, 0)

        def fire_scatter(j):
            s[j] = pltpu.async_copy(
                xpose_v.at[j % 2],
                out_hbm.at[
                    pl.ds(b0 // 128, b_per // 128),
                    pl.ds((f0 + j) * CDIM, CDIM),
                    :,
                ],
                ssem[j % 2],
            )

        fire_gather(0)
        for j in range(f_per):
            g[j].wait()
            if j + 1 < f_per:
                fire_gather(j + 1)
            if j >= 2:
                s[j - 2].wait()
            transpose_chunk(j)
            fire_scatter(j)
        s[f_per - 1].wait()
        s[f_per - 2].wait()

    return gather(xT, table)


def _mm_body(x_ref, w_ref, b_ref, out_ref):
    nf = out_ref.shape[0]
    mm = lax.dot_general(
        w_ref[...], x_ref[0], (((1,), (0,)), ((), ())),
        preferred_element_type=jnp.float32,
    )
    out_ref[...] = mm.reshape(nf, EDIM, 128) + b_ref[...]


def _tc_project(x_cmaj, w_big, b3):
    nq, nrow, _ = x_cmaj.shape
    nf = nrow // CDIM
    nbatch = nq * 128
    return pl.pallas_call(
        _mm_body,
        grid=(nq,),
        in_specs=[
            pl.BlockSpec((1, nrow, 128), lambda q: (q, 0, 0)),
            pl.BlockSpec((nf * EDIM, nrow), lambda q: (0, 0)),
            pl.BlockSpec((1, EDIM, 1), lambda q: (0, 0, 0)),
        ],
        out_specs=pl.BlockSpec((nf, EDIM, 128), lambda q: (0, 0, q)),
        out_shape=jax.ShapeDtypeStruct((nf, EDIM, nbatch), jnp.float32),
    )(x_cmaj, w_big, b3)


def kernel(x, table, W, b):
    batch, feat = x.shape
    xT = x.T
    t_flat = _sc_linearize(table.T)
    nrows = table.shape[0]
    tb2 = (nrows // 128) * 128
    t_flat = lax.dynamic_update_slice(
        t_flat, table[tb2:].reshape(-1), (tb2 * CDIM,)
    )
    t_lin = t_flat.reshape(table.shape)
    x_cmaj = _sc_gather(xT, t_lin)
    w_big = jnp.kron(jnp.eye(feat, dtype=jnp.float32), W)
    outp = _tc_project(x_cmaj, w_big, b.reshape(1, EDIM, 1))
    return jnp.transpose(outp, (2, 0, 1))
